# Initial kernel scaffold; baseline (speedup 1.0000x reference)
import jax
import jax.numpy as jnp
from jax.experimental import pallas as pl


def kernel(x, edge_index, Wf0, bf0, Wb0, bb0, Wf1, bf1, Wb1, bb1):
    raise NotImplementedError("write your pallas kernel here")


# speedup vs baseline: 4.4203x; 4.4203x over previous
import functools

import jax
import jax.numpy as jnp
from jax import lax
from jax.experimental import pallas as pl
from jax.experimental.pallas import tpu as pltpu
from jax.experimental.pallas import tpu_sc as plsc

N = 10000
E = 320000
D = 128
ALPHA = 0.5

K = 128
NCH = E // K
NT = 16
NROWCH = N // 128
NTAIL = N - NROWCH * 128

_mesh = plsc.VectorSubcoreMesh(core_axis_name="c", subcore_axis_name="s")


@functools.partial(
    pl.kernel,
    out_type=jax.ShapeDtypeStruct((2, N, D), jnp.float32),
    mesh=_mesh,
    scratch_types=[
        pltpu.VMEM((K,), jnp.int32),
        pltpu.VMEM((K, D), jnp.float32),
        pltpu.VMEM_SHARED((N, D), jnp.float32),
    ],
)
def _deg_kernel(edge_hbm, ones_hbm, zeros_hbm, deg_out, idx_v, ones_v, acc_sp):
    cid = lax.axis_index("c")
    sid = lax.axis_index("s")
    pltpu.sync_copy(ones_hbm, ones_v)

    for k in range(5):
        c = sid + k * NT

        @pl.when(c < NROWCH)
        def _():
            pltpu.sync_copy(zeros_hbm, acc_sp.at[pl.ds(c * 128, 128)])

    @pl.when(sid == 0)
    def _():
        pltpu.sync_copy(zeros_hbm.at[pl.ds(0, NTAIL)],
                        acc_sp.at[pl.ds(NROWCH * 128, NTAIL)])

    plsc.subcore_barrier()

    nk = jnp.where(sid < NCH - NT * (NCH // NT), 1 + NCH // NT, NCH // NT)

    def body(k, carry):
        c = sid + k * NT
        pltpu.sync_copy(edge_hbm.at[cid, pl.ds(c * K, K)], idx_v)
        pltpu.sync_copy(ones_v, acc_sp.at[idx_v], add=True)
        return carry

    lax.fori_loop(0, nk, body, 0)
    plsc.subcore_barrier()

    for k in range(5):
        c = sid + k * NT

        @pl.when(c < NROWCH)
        def _():
            pltpu.sync_copy(acc_sp.at[pl.ds(c * 128, 128)],
                            deg_out.at[cid, pl.ds(c * 128, 128)])

    @pl.when(sid == 0)
    def _():
        pltpu.sync_copy(acc_sp.at[pl.ds(NROWCH * 128, NTAIL)],
                        deg_out.at[cid, pl.ds(NROWCH * 128, NTAIL)])


@functools.partial(
    pl.kernel,
    out_type=(jax.ShapeDtypeStruct((N, D), jnp.float32),
              jax.ShapeDtypeStruct((N, D), jnp.float32)),
    mesh=_mesh,
    scratch_types=[
        pltpu.VMEM((K,), jnp.int32),
        pltpu.VMEM((K,), jnp.int32),
        pltpu.VMEM((K, D), jnp.float32),
        pltpu.VMEM_SHARED((N, D), jnp.float32),
        pltpu.SemaphoreType.DMA,
    ],
)
def _conv_kernel(mf_hbm, mb_hbm, edge_hbm, zeros_hbm, accf_out, accb_out,
                 gidx_v, sidx_v, buf_v, acc_sp, sem):
    cid = lax.axis_index("c")
    sid = lax.axis_index("s")

    def run(m_hbm, grow, srow, out_hbm):
        for k in range(5):
            c = sid + k * NT

            @pl.when(c < NROWCH)
            def _():
                pltpu.sync_copy(zeros_hbm, acc_sp.at[pl.ds(c * 128, 128)])

        @pl.when(sid == 0)
        def _():
            pltpu.sync_copy(zeros_hbm.at[pl.ds(0, NTAIL)],
                            acc_sp.at[pl.ds(NROWCH * 128, NTAIL)])

        plsc.subcore_barrier()

        nk = jnp.where(sid < NCH - NT * (NCH // NT), 1 + NCH // NT, NCH // NT)

        def body(k, carry):
            c = sid + k * NT
            off = c * K
            pltpu.sync_copy(edge_hbm.at[grow, pl.ds(off, K)], gidx_v)
            pltpu.sync_copy(edge_hbm.at[srow, pl.ds(off, K)], sidx_v)
            pltpu.async_copy(m_hbm.at[gidx_v], buf_v, sem).wait()
            pltpu.sync_copy(buf_v, acc_sp.at[sidx_v], add=True)
            return carry

        lax.fori_loop(0, nk, body, 0)
        plsc.subcore_barrier()

        for k in range(5):
            c = sid + k * NT

            @pl.when(c < NROWCH)
            def _():
                pltpu.sync_copy(acc_sp.at[pl.ds(c * 128, 128)],
                                out_hbm.at[pl.ds(c * 128, 128)])

        @pl.when(sid == 0)
        def _():
            pltpu.sync_copy(acc_sp.at[pl.ds(NROWCH * 128, NTAIL)],
                            out_hbm.at[pl.ds(NROWCH * 128, NTAIL)])

    @pl.when(cid == 0)
    def _():
        run(mf_hbm, 0, 1, accf_out)

    @pl.when(cid == 1)
    def _():
        run(mb_hbm, 1, 0, accb_out)


def _mm0_body(x_ref, deg_ref, wf_ref, wb_ref, mf_ref, mb_ref):
    x = x_ref[...]
    ds = lax.rsqrt(jnp.maximum(deg_ref[:, 0:1], 1.0))
    dd = lax.rsqrt(jnp.maximum(deg_ref[:, 1:2], 1.0))
    mf_ref[...] = jnp.dot(x * ds, wf_ref[...], preferred_element_type=jnp.float32)
    mb_ref[...] = jnp.dot(x * dd, wb_ref[...], preferred_element_type=jnp.float32)


def _mid_body(accf_ref, accb_ref, deg_ref, bf_ref, bb_ref, wf_ref, wb_ref,
              mf_ref, mb_ref):
    ds = lax.rsqrt(jnp.maximum(deg_ref[:, 0:1], 1.0))
    dd = lax.rsqrt(jnp.maximum(deg_ref[:, 1:2], 1.0))
    h = (ALPHA * (accf_ref[...] * dd + bf_ref[...])
         + (1.0 - ALPHA) * (accb_ref[...] * ds + bb_ref[...]))
    h = jnp.maximum(h, 0.0)
    mf_ref[...] = jnp.dot(h * ds, wf_ref[...], preferred_element_type=jnp.float32)
    mb_ref[...] = jnp.dot(h * dd, wb_ref[...], preferred_element_type=jnp.float32)


def _fin_body(accf_ref, accb_ref, deg_ref, bf_ref, bb_ref, out_ref):
    ds = lax.rsqrt(jnp.maximum(deg_ref[:, 0:1], 1.0))
    dd = lax.rsqrt(jnp.maximum(deg_ref[:, 1:2], 1.0))
    out_ref[...] = (ALPHA * (accf_ref[...] * dd + bf_ref[...])
                    + (1.0 - ALPHA) * (accb_ref[...] * ds + bb_ref[...]))


_f32 = jnp.float32
_nd = jax.ShapeDtypeStruct((N, D), _f32)


def kernel(x, edge_index, Wf0, bf0, Wb0, bb0, Wf1, bf1, Wb1, bb1):
    onesD = jnp.ones((K, D), _f32)
    zerosD = jnp.zeros((128, D), _f32)

    deg3 = _deg_kernel(edge_index, onesD, zerosD)
    deg = jnp.stack([deg3[0, :, 0], deg3[1, :, 0]], axis=1)

    mf0, mb0 = pl.pallas_call(_mm0_body, out_shape=(_nd, _nd))(
        x, deg, Wf0, Wb0)
    accf0, accb0 = _conv_kernel(mf0, mb0, edge_index, zerosD)

    mf1, mb1 = pl.pallas_call(_mid_body, out_shape=(_nd, _nd))(
        accf0, accb0, deg, bf0.reshape(1, D), bb0.reshape(1, D), Wf1, Wb1)
    accf1, accb1 = _conv_kernel(mf1, mb1, edge_index, zerosD)

    out = pl.pallas_call(_fin_body, out_shape=_nd)(
        accf1, accb1, deg, bf1.reshape(1, D), bb1.reshape(1, D))
    return out
